# baseline (speedup 1.0000x reference)
import functools

import jax
import jax.numpy as jnp
from jax import lax
from jax.experimental import pallas as pl
from jax.experimental.pallas import tpu as pltpu
from jax.experimental.pallas import tpu_sc as plsc

N = 10000
D_FEAT = 128
HIDDEN = 32
CODE = 16
E = 160000

NC = 2
NS = 16
NW = NC * NS
EPW = E // NW
CH = 125
NCH = EPW // CH
NP = 10240
RPS = NP // NS

def _mesh():
  return plsc.VectorSubcoreMesh(
      core_axis_name="c", subcore_axis_name="s", num_cores=NC, num_subcores=NS)


@functools.lru_cache(maxsize=None)
def _make_degree_kernel():
  @functools.partial(
      pl.kernel,
      out_type=jax.ShapeDtypeStruct((NC, NP, 16), jnp.float32),
      mesh=_mesh(),
      scratch_types=[
          pltpu.VMEM((NCH, CH), jnp.int32),
          pltpu.VMEM((CH, 16), jnp.float32),
          pltpu.VMEM_SHARED((NP, 16), jnp.float32),
          pltpu.SemaphoreType.DMA,
      ],
      compiler_params=pltpu.CompilerParams(use_tc_tiling_on_sc=False),
  )
  def k(dst_hbm, ones_hbm, zero_hbm, out_hbm, dst_v, ones_v, acc, sem):
    cid = lax.axis_index("c")
    sid = lax.axis_index("s")
    wid = cid * NS + sid
    pltpu.sync_copy(zero_hbm.at[pl.ds(sid * RPS, RPS)],
                    acc.at[pl.ds(sid * RPS, RPS)])
    pltpu.sync_copy(dst_hbm.at[wid], dst_v)
    pltpu.sync_copy(ones_hbm, ones_v)
    plsc.subcore_barrier()

    def fire(j, carry):
      pltpu.async_copy(ones_v, acc.at[dst_v.at[j]], sem, add=True)
      return carry

    lax.fori_loop(0, NCH, fire, 0)

    def drain(j, carry):
      pltpu.make_async_copy(zero_hbm.at[pl.ds(0, CH)], ones_v, sem).wait()
      return carry

    lax.fori_loop(0, NCH, drain, 0)
    plsc.subcore_barrier()
    pltpu.sync_copy(acc.at[pl.ds(sid * RPS, RPS)],
                    out_hbm.at[cid, pl.ds(sid * RPS, RPS)])

  return k


@functools.lru_cache(maxsize=None)
def _make_segsum_kernel(F):
  @functools.partial(
      pl.kernel,
      out_type=jax.ShapeDtypeStruct((NC, NP, F), jnp.float32),
      mesh=_mesh(),
      scratch_types=[
          pltpu.VMEM((NCH, CH), jnp.int32),
          pltpu.VMEM((NCH, CH), jnp.int32),
          pltpu.VMEM((CH, F), jnp.float32),
          pltpu.VMEM((CH, F), jnp.float32),
          pltpu.VMEM((CH, F), jnp.float32),
          pltpu.VMEM((CH, F), jnp.float32),
          pltpu.VMEM_SHARED((NP, F), jnp.float32),
          pltpu.SemaphoreType.DMA,
          pltpu.SemaphoreType.DMA,
      ],
      compiler_params=pltpu.CompilerParams(use_tc_tiling_on_sc=False),
  )
  def k(g_hbm, src_hbm, dst_hbm, zero_hbm, out_hbm,
        src_v, dst_v, rows0, rows1, rows2, rows3, acc, gsem, ssem):
    cid = lax.axis_index("c")
    sid = lax.axis_index("s")
    wid = cid * NS + sid
    pltpu.sync_copy(zero_hbm.at[pl.ds(sid * RPS, RPS)],
                    acc.at[pl.ds(sid * RPS, RPS)])
    pltpu.sync_copy(src_hbm.at[wid], src_v)
    pltpu.sync_copy(dst_hbm.at[wid], dst_v)
    plsc.subcore_barrier()

    bufs = (rows0, rows1, rows2, rows3)

    def gstart(j, buf):
      pltpu.async_copy(g_hbm.at[src_v.at[j]], buf, gsem)

    def gwait(buf):
      pltpu.make_async_copy(g_hbm.at[pl.ds(0, CH)], buf, gsem).wait()

    def scat(j, buf):
      pltpu.sync_copy(buf, acc.at[dst_v.at[j]], add=True)

    gstart(0, bufs[0])
    gstart(1, bufs[1])
    gstart(2, bufs[2])
    gwait(bufs[0])
    gstart(3, bufs[3])
    scat(0, bufs[0])

    def quad(k4, carry):
      j0 = 4 * k4
      for b in (1, 2, 3, 0):
        j = j0 + b if b else j0 + 4
        gwait(bufs[b])
        gstart(j + 3, bufs[(b + 3) % 4])
        scat(j, bufs[b])
      return carry

    lax.fori_loop(0, (NCH - 4) // 4, quad, 0)
    for j, b in ((NCH - 3, 1), (NCH - 2, 2), (NCH - 1, 3)):
      gwait(bufs[b])
      scat(j, bufs[b])
    plsc.subcore_barrier()
    pltpu.sync_copy(acc.at[pl.ds(sid * RPS, RPS)],
                    out_hbm.at[cid, pl.ds(sid * RPS, RPS)])

  return k


_RB = 1000


@functools.lru_cache(maxsize=None)
def _make_fused_layer1_kernel():
  @functools.partial(
      pl.kernel,
      out_type=(jax.ShapeDtypeStruct((NC, NP, HIDDEN), jnp.float32),
                jax.ShapeDtypeStruct((NC, NP, 16), jnp.float32)),
      mesh=_mesh(),
      scratch_types=[
          pltpu.VMEM((2 * NCH, CH), jnp.int32),
          pltpu.VMEM((CH, 16), jnp.float32),
          pltpu.VMEM((NCH, CH), jnp.int32),
          pltpu.VMEM((NCH, CH), jnp.int32),
          pltpu.VMEM((RPS, 16), jnp.float32),
          pltpu.VMEM((RPS, 16), jnp.float32),
          pltpu.VMEM((RPS, HIDDEN), jnp.float32),
          pltpu.VMEM((CH, HIDDEN), jnp.float32),
          pltpu.VMEM((CH, HIDDEN), jnp.float32),
          pltpu.VMEM((CH, HIDDEN), jnp.float32),
          pltpu.VMEM((CH, HIDDEN), jnp.float32),
          pltpu.VMEM_SHARED((NP, 16), jnp.float32),
          pltpu.VMEM_SHARED((NP, HIDDEN), jnp.float32),
          pltpu.VMEM_SHARED((NP, HIDDEN), jnp.float32),
          pltpu.SemaphoreType.DMA,
          pltpu.SemaphoreType.DMA,
      ],
      compiler_params=pltpu.CompilerParams(use_tc_tiling_on_sc=False,
                                           needs_layout_passes=False),
  )
  def k(hw1_hbm, src_hbm, dst_hbm, ones_hbm, zero16_hbm, zero32_hbm,
        p_out, dinv_out,
        dst2_v, ones_v, src_v, dst_v, deg_v, dinv_v, hw_v,
        rows0, rows1, rows2, rows3, acc16, g1t, acc32, dsem, gsem):
    cid = lax.axis_index("c")
    sid = lax.axis_index("s")
    wid = cid * NS + sid
    row0 = sid * RPS

    pltpu.sync_copy(zero16_hbm.at[pl.ds(row0, RPS)], acc16.at[pl.ds(row0, RPS)])
    pltpu.sync_copy(dst_hbm.at[sid], dst2_v.at[pl.ds(0, NCH)])
    pltpu.sync_copy(dst_hbm.at[sid + NS], dst2_v.at[pl.ds(NCH, NCH)])
    pltpu.sync_copy(ones_hbm, ones_v)
    plsc.subcore_barrier()

    def fire(j, carry):
      pltpu.async_copy(ones_v, acc16.at[dst2_v.at[j]], dsem, add=True)
      return carry

    lax.fori_loop(0, 2 * NCH, fire, 0)

    def drain(j, carry):
      pltpu.make_async_copy(zero16_hbm.at[pl.ds(0, CH)], ones_v, dsem).wait()
      return carry

    lax.fori_loop(0, 2 * NCH, drain, 0)
    plsc.subcore_barrier()

    pltpu.sync_copy(acc16.at[pl.ds(row0, RPS)], deg_v)
    pltpu.sync_copy(hw1_hbm.at[pl.ds(row0, RPS)], hw_v)

    def nrow(r, carry):
      xdeg = deg_v[r] + 1.0
      i = plsc.bitcast(xdeg, jnp.int32)
      i = 0x5F3759DF - lax.shift_right_logical(i, 1)
      y = plsc.bitcast(i, jnp.float32)
      y = y * (1.5 - 0.5 * xdeg * y * y)
      y = y * (1.5 - 0.5 * xdeg * y * y)
      y = y * (1.5 - 0.5 * xdeg * y * y)
      dinv_v[r] = y
      hw_v[r, pl.ds(0, 16)] = hw_v[r, pl.ds(0, 16)] * y
      hw_v[r, pl.ds(16, 16)] = hw_v[r, pl.ds(16, 16)] * y
      return carry

    lax.fori_loop(0, RPS, nrow, 0)
    pltpu.sync_copy(dinv_v, dinv_out.at[cid, pl.ds(row0, RPS)])
    pltpu.sync_copy(hw_v, g1t.at[pl.ds(row0, RPS)])

    @pl.when(cid == 0)
    def _():
      pltpu.sync_copy(hw_v, acc32.at[pl.ds(row0, RPS)])

    @pl.when(cid == 1)
    def _():
      pltpu.sync_copy(zero32_hbm.at[pl.ds(row0, RPS)],
                      acc32.at[pl.ds(row0, RPS)])

    pltpu.sync_copy(src_hbm.at[wid], src_v)
    pltpu.sync_copy(dst_hbm.at[wid], dst_v)
    plsc.subcore_barrier()

    bufs = (rows0, rows1, rows2, rows3)

    def gstart(j, buf):
      pltpu.async_copy(g1t.at[src_v.at[j]], buf, gsem)

    def gwait(buf):
      pltpu.make_async_copy(zero32_hbm.at[pl.ds(0, CH)], buf, gsem).wait()

    def scat(j, buf):
      pltpu.sync_copy(buf, acc32.at[dst_v.at[j]], add=True)

    gstart(0, bufs[0])
    gstart(1, bufs[1])
    gstart(2, bufs[2])
    gwait(bufs[0])
    gstart(3, bufs[3])
    scat(0, bufs[0])

    def quad(k4, carry):
      j0 = 4 * k4
      for b in (1, 2, 3, 0):
        j = j0 + b if b else j0 + 4
        gwait(bufs[b])
        gstart(j + 3, bufs[(b + 3) % 4])
        scat(j, bufs[b])
      return carry

    lax.fori_loop(0, (NCH - 4) // 4, quad, 0)
    for j, b in ((NCH - 3, 1), (NCH - 2, 2), (NCH - 1, 3)):
      gwait(bufs[b])
      scat(j, bufs[b])
    plsc.subcore_barrier()
    pltpu.sync_copy(acc32.at[pl.ds(row0, RPS)],
                    p_out.at[cid, pl.ds(row0, RPS)])

  return k


def _mat1_body(x_ref, w1_ref, out_ref):
  out_ref[...] = jnp.dot(x_ref[...], w1_ref[...],
                         preferred_element_type=jnp.float32)


def _prep2_body(dinv_ref, p1_ref, w2_ref, out_ref):
  dinv = dinv_ref[0, :, 0]
  h = jnp.maximum((p1_ref[0] + p1_ref[1]) * dinv[:, None], 0.0)
  g2 = jnp.dot(h, w2_ref[...], preferred_element_type=jnp.float32)
  out_ref[...] = g2 * dinv[:, None]


def _enc_body(dinv_ref, g2_ref, p2_ref, out_ref):
  dinv = dinv_ref[0, :, 0]
  out_ref[...] = (g2_ref[...] + p2_ref[0] + p2_ref[1]) * dinv[:, None]


_BM = 200


def _dec_body(ei_ref, ej_ref, out_ref):
  z = lax.dot_general(ei_ref[...], ej_ref[...], (((1,), (1,)), ((), ())),
                      preferred_element_type=jnp.float32)
  out_ref[...] = 0.5 * jnp.tanh(z * 0.5) + 0.5


def kernel(x, edge_index, W1, W2):
  src3 = edge_index[0].reshape(NW, NCH, CH)
  dst3 = edge_index[1].reshape(NW, NCH, CH)
  zeros16 = jnp.zeros((NP, 16), jnp.float32)
  zeros32 = jnp.zeros((NP, HIDDEN), jnp.float32)
  ones = jnp.ones((CH, 16), jnp.float32)

  hw1 = pl.pallas_call(
      _mat1_body,
      grid=(N // _RB,),
      in_specs=[
          pl.BlockSpec((_RB, D_FEAT), lambda i: (i, 0)),
          pl.BlockSpec((D_FEAT, HIDDEN), lambda i: (0, 0)),
      ],
      out_specs=pl.BlockSpec((_RB, HIDDEN), lambda i: (i, 0)),
      out_shape=jax.ShapeDtypeStruct((N, HIDDEN), jnp.float32),
  )(x, W1)

  p1, dinv16 = _make_fused_layer1_kernel()(
      hw1, src3, dst3, ones, zeros16, zeros32)

  g2 = pl.pallas_call(
      _prep2_body,
      grid=(N // _RB,),
      in_specs=[
          pl.BlockSpec((NC, _RB, 16), lambda i: (0, i, 0)),
          pl.BlockSpec((NC, _RB, HIDDEN), lambda i: (0, i, 0)),
          pl.BlockSpec((HIDDEN, CODE), lambda i: (0, 0)),
      ],
      out_specs=pl.BlockSpec((_RB, CODE), lambda i: (i, 0)),
      out_shape=jax.ShapeDtypeStruct((N, CODE), jnp.float32),
  )(dinv16, p1, W2)

  p2 = _make_segsum_kernel(CODE)(g2, src3, dst3, zeros16)

  encoded = pl.pallas_call(
      _enc_body,
      grid=(N // _RB,),
      in_specs=[
          pl.BlockSpec((NC, _RB, 16), lambda i: (0, i, 0)),
          pl.BlockSpec((_RB, CODE), lambda i: (i, 0)),
          pl.BlockSpec((NC, _RB, CODE), lambda i: (0, i, 0)),
      ],
      out_specs=pl.BlockSpec((_RB, CODE), lambda i: (i, 0)),
      out_shape=jax.ShapeDtypeStruct((N, CODE), jnp.float32),
  )(dinv16, g2, p2)

  prediction = pl.pallas_call(
      _dec_body,
      grid=(N // _BM,),
      in_specs=[
          pl.BlockSpec((_BM, CODE), lambda i: (i, 0)),
          pl.BlockSpec((N, CODE), lambda i: (0, 0)),
      ],
      out_specs=pl.BlockSpec((_BM, N), lambda i: (i, 0)),
      out_shape=jax.ShapeDtypeStruct((N, N), jnp.float32),
      compiler_params=pltpu.CompilerParams(
          dimension_semantics=("arbitrary",)),
  )(encoded, encoded)

  return prediction


# speedup vs baseline: 1.0360x; 1.0360x over previous
import functools

import jax
import jax.numpy as jnp
from jax import lax
from jax.experimental import pallas as pl
from jax.experimental.pallas import tpu as pltpu
from jax.experimental.pallas import tpu_sc as plsc

N = 10000
D_FEAT = 128
HIDDEN = 32
CODE = 16
E = 160000

NC = 2
NS = 16
NW = NC * NS
EPW = E // NW
CH = 125
NCH = EPW // CH
NP = 10240
RPS = NP // NS

def _mesh():
  return plsc.VectorSubcoreMesh(
      core_axis_name="c", subcore_axis_name="s", num_cores=NC, num_subcores=NS)


@functools.lru_cache(maxsize=None)
def _make_degree_kernel():
  @functools.partial(
      pl.kernel,
      out_type=jax.ShapeDtypeStruct((NC, NP, 16), jnp.float32),
      mesh=_mesh(),
      scratch_types=[
          pltpu.VMEM((NCH, CH), jnp.int32),
          pltpu.VMEM((CH, 16), jnp.float32),
          pltpu.VMEM_SHARED((NP, 16), jnp.float32),
          pltpu.SemaphoreType.DMA,
      ],
      compiler_params=pltpu.CompilerParams(use_tc_tiling_on_sc=False),
  )
  def k(dst_hbm, ones_hbm, zero_hbm, out_hbm, dst_v, ones_v, acc, sem):
    cid = lax.axis_index("c")
    sid = lax.axis_index("s")
    wid = cid * NS + sid
    pltpu.sync_copy(zero_hbm.at[pl.ds(sid * RPS, RPS)],
                    acc.at[pl.ds(sid * RPS, RPS)])
    pltpu.sync_copy(dst_hbm.at[wid], dst_v)
    pltpu.sync_copy(ones_hbm, ones_v)
    plsc.subcore_barrier()

    def fire(j, carry):
      pltpu.async_copy(ones_v, acc.at[dst_v.at[j]], sem, add=True)
      return carry

    lax.fori_loop(0, NCH, fire, 0)

    def drain(j, carry):
      pltpu.make_async_copy(zero_hbm.at[pl.ds(0, CH)], ones_v, sem).wait()
      return carry

    lax.fori_loop(0, NCH, drain, 0)
    plsc.subcore_barrier()
    pltpu.sync_copy(acc.at[pl.ds(sid * RPS, RPS)],
                    out_hbm.at[cid, pl.ds(sid * RPS, RPS)])

  return k


@functools.lru_cache(maxsize=None)
def _make_segsum_kernel(F):
  @functools.partial(
      pl.kernel,
      out_type=jax.ShapeDtypeStruct((NC, NP, F), jnp.float32),
      mesh=_mesh(),
      scratch_types=[
          pltpu.VMEM((NCH, CH), jnp.int32),
          pltpu.VMEM((NCH, CH), jnp.int32),
          pltpu.VMEM((CH, F), jnp.float32),
          pltpu.VMEM((CH, F), jnp.float32),
          pltpu.VMEM((CH, F), jnp.float32),
          pltpu.VMEM((CH, F), jnp.float32),
          pltpu.VMEM_SHARED((NP, F), jnp.float32),
          pltpu.SemaphoreType.DMA,
          pltpu.SemaphoreType.DMA,
      ],
      compiler_params=pltpu.CompilerParams(use_tc_tiling_on_sc=False),
  )
  def k(g_hbm, src_hbm, dst_hbm, zero_hbm, out_hbm,
        src_v, dst_v, rows0, rows1, rows2, rows3, acc, gsem, ssem):
    cid = lax.axis_index("c")
    sid = lax.axis_index("s")
    wid = cid * NS + sid
    pltpu.sync_copy(zero_hbm.at[pl.ds(sid * RPS, RPS)],
                    acc.at[pl.ds(sid * RPS, RPS)])
    pltpu.sync_copy(src_hbm.at[wid], src_v)
    pltpu.sync_copy(dst_hbm.at[wid], dst_v)
    plsc.subcore_barrier()

    bufs = (rows0, rows1, rows2, rows3)

    def gstart(j, buf):
      pltpu.async_copy(g_hbm.at[src_v.at[j]], buf, gsem)

    def gwait(buf):
      pltpu.make_async_copy(g_hbm.at[pl.ds(0, CH)], buf, gsem).wait()

    def scat(j, buf):
      pltpu.sync_copy(buf, acc.at[dst_v.at[j]], add=True)

    gstart(0, bufs[0])
    gstart(1, bufs[1])
    gstart(2, bufs[2])
    gwait(bufs[0])
    gstart(3, bufs[3])
    scat(0, bufs[0])

    def quad(k4, carry):
      j0 = 4 * k4
      for b in (1, 2, 3, 0):
        j = j0 + b if b else j0 + 4
        gwait(bufs[b])
        gstart(j + 3, bufs[(b + 3) % 4])
        scat(j, bufs[b])
      return carry

    lax.fori_loop(0, (NCH - 4) // 4, quad, 0)
    for j, b in ((NCH - 3, 1), (NCH - 2, 2), (NCH - 1, 3)):
      gwait(bufs[b])
      scat(j, bufs[b])
    plsc.subcore_barrier()
    pltpu.sync_copy(acc.at[pl.ds(sid * RPS, RPS)],
                    out_hbm.at[cid, pl.ds(sid * RPS, RPS)])

  return k


_RB = 1000


@functools.lru_cache(maxsize=None)
def _make_fused_layer1_kernel():
  @functools.partial(
      pl.kernel,
      out_type=(jax.ShapeDtypeStruct((NC, NP, HIDDEN), jnp.float32),
                jax.ShapeDtypeStruct((NC, NP, 16), jnp.float32),
                jax.ShapeDtypeStruct((NC, NP, HIDDEN), jnp.float32),
                jax.ShapeDtypeStruct((NC, NP, HIDDEN), jnp.float32)),
      mesh=_mesh(),
      scratch_types=[
          pltpu.VMEM((2 * NCH, CH), jnp.int32),
          pltpu.VMEM((CH, 16), jnp.float32),
          pltpu.VMEM((NCH, CH), jnp.int32),
          pltpu.VMEM((NCH, CH), jnp.int32),
          pltpu.VMEM((RPS, 16), jnp.float32),
          pltpu.VMEM((RPS // 2, HIDDEN), jnp.float32),
          pltpu.VMEM((RPS, HIDDEN), jnp.float32),
          pltpu.VMEM((CH, HIDDEN), jnp.float32),
          pltpu.VMEM((CH, HIDDEN), jnp.float32),
          pltpu.VMEM((CH, HIDDEN), jnp.float32),
          pltpu.VMEM((CH, HIDDEN), jnp.float32),
          pltpu.VMEM_SHARED((NP, 16), jnp.float32),
          pltpu.VMEM_SHARED((NP, HIDDEN), jnp.float32),
          pltpu.VMEM_SHARED((NP, HIDDEN), jnp.float32),
          pltpu.SemaphoreType.DMA,
          pltpu.SemaphoreType.DMA,
          pltpu.SemaphoreType.REGULAR,
      ],
      compiler_params=pltpu.CompilerParams(use_tc_tiling_on_sc=False,
                                           needs_layout_passes=False),
  )
  def k(hw1_hbm, src_hbm, dst_hbm, ones_hbm, zero16_hbm, zero32_hbm,
        p_out, dinv_out, hp_out, q_out,
        dst2_v, ones_v, src_v, dst_v, deg_v, pb2_v, hw_v,
        rows0, rows1, rows2, rows3, acc16, g1t, acc32,
        dsem, gsem, xsem):
    cid = lax.axis_index("c")
    sid = lax.axis_index("s")
    wid = cid * NS + sid
    row0 = sid * RPS

    pltpu.sync_copy(zero16_hbm.at[pl.ds(row0, RPS)], acc16.at[pl.ds(row0, RPS)])
    pltpu.sync_copy(dst_hbm.at[sid], dst2_v.at[pl.ds(0, NCH)])
    pltpu.sync_copy(dst_hbm.at[sid + NS], dst2_v.at[pl.ds(NCH, NCH)])
    pltpu.sync_copy(ones_hbm, ones_v)
    plsc.subcore_barrier()

    def fire(j, carry):
      pltpu.async_copy(ones_v, acc16.at[dst2_v.at[j]], dsem, add=True)
      return carry

    lax.fori_loop(0, 2 * NCH, fire, 0)

    def drain(j, carry):
      pltpu.make_async_copy(zero16_hbm.at[pl.ds(0, CH)], ones_v, dsem).wait()
      return carry

    lax.fori_loop(0, 2 * NCH, drain, 0)
    plsc.subcore_barrier()

    pltpu.sync_copy(acc16.at[pl.ds(row0, RPS)], deg_v)
    pltpu.sync_copy(hw1_hbm.at[pl.ds(row0, RPS)], hw_v)

    def nrow(r, carry):
      xdeg = deg_v[r] + 1.0
      i = plsc.bitcast(xdeg, jnp.int32)
      i = 0x5F3759DF - lax.shift_right_logical(i, 1)
      y = plsc.bitcast(i, jnp.float32)
      y = y * (1.5 - 0.5 * xdeg * y * y)
      y = y * (1.5 - 0.5 * xdeg * y * y)
      y = y * (1.5 - 0.5 * xdeg * y * y)
      deg_v[r] = y
      hw_v[r, pl.ds(0, 16)] = hw_v[r, pl.ds(0, 16)] * y
      hw_v[r, pl.ds(16, 16)] = hw_v[r, pl.ds(16, 16)] * y
      return carry

    lax.fori_loop(0, RPS, nrow, 0)
    pltpu.sync_copy(deg_v, dinv_out.at[cid, pl.ds(row0, RPS)])
    pltpu.sync_copy(hw_v, g1t.at[pl.ds(row0, RPS)])

    @pl.when(cid == 0)
    def _():
      pltpu.sync_copy(hw_v, acc32.at[pl.ds(row0, RPS)])

    @pl.when(cid == 1)
    def _():
      pltpu.sync_copy(zero32_hbm.at[pl.ds(row0, RPS)],
                      acc32.at[pl.ds(row0, RPS)])

    pltpu.sync_copy(src_hbm.at[wid], src_v)
    pltpu.sync_copy(dst_hbm.at[wid], dst_v)
    plsc.subcore_barrier()

    bufs = (rows0, rows1, rows2, rows3)

    def gstart(j, buf):
      pltpu.async_copy(g1t.at[src_v.at[j]], buf, gsem)

    def gwait(buf):
      pltpu.make_async_copy(zero32_hbm.at[pl.ds(0, CH)], buf, gsem).wait()

    def scat(j, buf):
      pltpu.sync_copy(buf, acc32.at[dst_v.at[j]], add=True)

    def segsum_pass():
      gstart(0, bufs[0])
      gstart(1, bufs[1])
      gstart(2, bufs[2])
      gwait(bufs[0])
      gstart(3, bufs[3])
      scat(0, bufs[0])

      def quad(k4, carry):
        j0 = 4 * k4
        for b in (1, 2, 3, 0):
          j = j0 + b if b else j0 + 4
          gwait(bufs[b])
          gstart(j + 3, bufs[(b + 3) % 4])
          scat(j, bufs[b])
        return carry

      lax.fori_loop(0, (NCH - 4) // 4, quad, 0)
      for j, b in ((NCH - 3, 1), (NCH - 2, 2), (NCH - 1, 3)):
        gwait(bufs[b])
        scat(j, bufs[b])

    segsum_pass()
    plsc.subcore_barrier()
    pltpu.sync_copy(acc32.at[pl.ds(row0, RPS)],
                    p_out.at[cid, pl.ds(row0, RPS)])

    plsc.subcore_barrier()

    @pl.when(sid == 0)
    def _():
      pl.semaphore_signal(xsem, 1, core_index=1 - cid)
      pl.semaphore_wait(xsem, 1)

    plsc.subcore_barrier()
    pltpu.sync_copy(p_out.at[0, pl.ds(row0, RPS)], hw_v)
    HALF = RPS // 2
    for h in (0, 1):
      pltpu.sync_copy(p_out.at[1, pl.ds(row0 + h * HALF, HALF)], pb2_v)

      def hrow(r, carry, _h=h):
        rr = _h * HALF + r
        y = deg_v[rr]
        for c in (0, 16):
          srow = (hw_v[rr, pl.ds(c, 16)] + pb2_v[r, pl.ds(c, 16)]) * y
          hw_v[rr, pl.ds(c, 16)] = jnp.maximum(srow, 0.0) * y
        return carry

      lax.fori_loop(0, HALF, hrow, 0)
    pltpu.sync_copy(hw_v, hp_out.at[cid, pl.ds(row0, RPS)])
    pltpu.sync_copy(hw_v, g1t.at[pl.ds(row0, RPS)])
    pltpu.sync_copy(zero32_hbm.at[pl.ds(row0, RPS)],
                    acc32.at[pl.ds(row0, RPS)])
    plsc.subcore_barrier()

    segsum_pass()
    plsc.subcore_barrier()
    pltpu.sync_copy(acc32.at[pl.ds(row0, RPS)],
                    q_out.at[cid, pl.ds(row0, RPS)])

  return k


def _mat1_body(x_ref, w1_ref, out_ref):
  out_ref[...] = jnp.dot(x_ref[...], w1_ref[...],
                         preferred_element_type=jnp.float32)


def _enc2_body(dinv_ref, q_ref, hp_ref, w2_ref, out_ref):
  dinv = dinv_ref[0, :, 0]
  s = (q_ref[0] + q_ref[1] + hp_ref[0]) * dinv[:, None]
  out_ref[...] = jnp.dot(s, w2_ref[...], preferred_element_type=jnp.float32)


_BM = 200


def _dec_body(ei_ref, ej_ref, out_ref):
  z = lax.dot_general(ei_ref[...], ej_ref[...], (((1,), (1,)), ((), ())),
                      preferred_element_type=jnp.float32)
  out_ref[...] = 0.5 * jnp.tanh(z * 0.5) + 0.5


def kernel(x, edge_index, W1, W2):
  src3 = edge_index[0].reshape(NW, NCH, CH)
  dst3 = edge_index[1].reshape(NW, NCH, CH)
  zeros16 = jnp.zeros((NP, 16), jnp.float32)
  zeros32 = jnp.zeros((NP, HIDDEN), jnp.float32)
  ones = jnp.ones((CH, 16), jnp.float32)

  hw1 = pl.pallas_call(
      _mat1_body,
      grid=(N // _RB,),
      in_specs=[
          pl.BlockSpec((_RB, D_FEAT), lambda i: (i, 0)),
          pl.BlockSpec((D_FEAT, HIDDEN), lambda i: (0, 0)),
      ],
      out_specs=pl.BlockSpec((_RB, HIDDEN), lambda i: (i, 0)),
      out_shape=jax.ShapeDtypeStruct((N, HIDDEN), jnp.float32),
  )(x, W1)

  p1, dinv16, hp, q = _make_fused_layer1_kernel()(
      hw1, src3, dst3, ones, zeros16, zeros32)

  encoded = pl.pallas_call(
      _enc2_body,
      grid=(N // _RB,),
      in_specs=[
          pl.BlockSpec((NC, _RB, 16), lambda i: (0, i, 0)),
          pl.BlockSpec((NC, _RB, HIDDEN), lambda i: (0, i, 0)),
          pl.BlockSpec((NC, _RB, HIDDEN), lambda i: (0, i, 0)),
          pl.BlockSpec((HIDDEN, CODE), lambda i: (0, 0)),
      ],
      out_specs=pl.BlockSpec((_RB, CODE), lambda i: (i, 0)),
      out_shape=jax.ShapeDtypeStruct((N, CODE), jnp.float32),
  )(dinv16, q, hp, W2)

  prediction = pl.pallas_call(
      _dec_body,
      grid=(N // _BM,),
      in_specs=[
          pl.BlockSpec((_BM, CODE), lambda i: (i, 0)),
          pl.BlockSpec((N, CODE), lambda i: (0, 0)),
      ],
      out_specs=pl.BlockSpec((_BM, N), lambda i: (i, 0)),
      out_shape=jax.ShapeDtypeStruct((N, N), jnp.float32),
      compiler_params=pltpu.CompilerParams(
          dimension_semantics=("arbitrary",)),
  )(encoded, encoded)

  return prediction


# speedup vs baseline: 1.0387x; 1.0026x over previous
import functools

import jax
import jax.numpy as jnp
from jax import lax
from jax.experimental import pallas as pl
from jax.experimental.pallas import tpu as pltpu
from jax.experimental.pallas import tpu_sc as plsc

N = 10000
D_FEAT = 128
HIDDEN = 32
CODE = 16
E = 160000

NC = 2
NS = 16
NW = NC * NS
EPW = E // NW
CH = 125
NCH = EPW // CH
NP = 10240
RPS = NP // NS

def _mesh():
  return plsc.VectorSubcoreMesh(
      core_axis_name="c", subcore_axis_name="s", num_cores=NC, num_subcores=NS)


@functools.lru_cache(maxsize=None)
def _make_degree_kernel():
  @functools.partial(
      pl.kernel,
      out_type=jax.ShapeDtypeStruct((NC, NP, 16), jnp.float32),
      mesh=_mesh(),
      scratch_types=[
          pltpu.VMEM((NCH, CH), jnp.int32),
          pltpu.VMEM((CH, 16), jnp.float32),
          pltpu.VMEM_SHARED((NP, 16), jnp.float32),
          pltpu.SemaphoreType.DMA,
      ],
      compiler_params=pltpu.CompilerParams(use_tc_tiling_on_sc=False),
  )
  def k(dst_hbm, ones_hbm, zero_hbm, out_hbm, dst_v, ones_v, acc, sem):
    cid = lax.axis_index("c")
    sid = lax.axis_index("s")
    wid = cid * NS + sid
    pltpu.sync_copy(zero_hbm.at[pl.ds(sid * RPS, RPS)],
                    acc.at[pl.ds(sid * RPS, RPS)])
    pltpu.sync_copy(dst_hbm.at[wid], dst_v)
    pltpu.sync_copy(ones_hbm, ones_v)
    plsc.subcore_barrier()

    def fire(j, carry):
      pltpu.async_copy(ones_v, acc.at[dst_v.at[j]], sem, add=True)
      return carry

    lax.fori_loop(0, NCH, fire, 0)

    def drain(j, carry):
      pltpu.make_async_copy(zero_hbm.at[pl.ds(0, CH)], ones_v, sem).wait()
      return carry

    lax.fori_loop(0, NCH, drain, 0)
    plsc.subcore_barrier()
    pltpu.sync_copy(acc.at[pl.ds(sid * RPS, RPS)],
                    out_hbm.at[cid, pl.ds(sid * RPS, RPS)])

  return k


@functools.lru_cache(maxsize=None)
def _make_segsum_kernel(F):
  @functools.partial(
      pl.kernel,
      out_type=jax.ShapeDtypeStruct((NC, NP, F), jnp.float32),
      mesh=_mesh(),
      scratch_types=[
          pltpu.VMEM((NCH, CH), jnp.int32),
          pltpu.VMEM((NCH, CH), jnp.int32),
          pltpu.VMEM((CH, F), jnp.float32),
          pltpu.VMEM((CH, F), jnp.float32),
          pltpu.VMEM((CH, F), jnp.float32),
          pltpu.VMEM((CH, F), jnp.float32),
          pltpu.VMEM_SHARED((NP, F), jnp.float32),
          pltpu.SemaphoreType.DMA,
          pltpu.SemaphoreType.DMA,
      ],
      compiler_params=pltpu.CompilerParams(use_tc_tiling_on_sc=False),
  )
  def k(g_hbm, src_hbm, dst_hbm, zero_hbm, out_hbm,
        src_v, dst_v, rows0, rows1, rows2, rows3, acc, gsem, ssem):
    cid = lax.axis_index("c")
    sid = lax.axis_index("s")
    wid = cid * NS + sid
    pltpu.sync_copy(zero_hbm.at[pl.ds(sid * RPS, RPS)],
                    acc.at[pl.ds(sid * RPS, RPS)])
    pltpu.sync_copy(src_hbm.at[wid], src_v)
    pltpu.sync_copy(dst_hbm.at[wid], dst_v)
    plsc.subcore_barrier()

    bufs = (rows0, rows1, rows2, rows3)

    def gstart(j, buf):
      pltpu.async_copy(g_hbm.at[src_v.at[j]], buf, gsem)

    def gwait(buf):
      pltpu.make_async_copy(g_hbm.at[pl.ds(0, CH)], buf, gsem).wait()

    def scat(j, buf):
      pltpu.sync_copy(buf, acc.at[dst_v.at[j]], add=True)

    gstart(0, bufs[0])
    gstart(1, bufs[1])
    gstart(2, bufs[2])
    gwait(bufs[0])
    gstart(3, bufs[3])
    scat(0, bufs[0])

    def quad(k4, carry):
      j0 = 4 * k4
      for b in (1, 2, 3, 0):
        j = j0 + b if b else j0 + 4
        gwait(bufs[b])
        gstart(j + 3, bufs[(b + 3) % 4])
        scat(j, bufs[b])
      return carry

    lax.fori_loop(0, (NCH - 4) // 4, quad, 0)
    for j, b in ((NCH - 3, 1), (NCH - 2, 2), (NCH - 1, 3)):
      gwait(bufs[b])
      scat(j, bufs[b])
    plsc.subcore_barrier()
    pltpu.sync_copy(acc.at[pl.ds(sid * RPS, RPS)],
                    out_hbm.at[cid, pl.ds(sid * RPS, RPS)])

  return k


_RB = 1000


@functools.lru_cache(maxsize=None)
def _make_fused_layer1_kernel():
  @functools.partial(
      pl.kernel,
      out_type=(jax.ShapeDtypeStruct((NC, NP, HIDDEN), jnp.float32),
                jax.ShapeDtypeStruct((NC, NP, 16), jnp.float32),
                jax.ShapeDtypeStruct((NC, NP, HIDDEN), jnp.float32),
                jax.ShapeDtypeStruct((NC, NP, HIDDEN), jnp.float32)),
      mesh=_mesh(),
      scratch_types=[
          pltpu.VMEM((RPS, 16), jnp.float32),
          pltpu.VMEM((CH, 16), jnp.float32),
          pltpu.VMEM((NCH, CH), jnp.int32),
          pltpu.VMEM((NCH, CH), jnp.int32),
          pltpu.VMEM((RPS, 16), jnp.float32),
          pltpu.VMEM((RPS // 2, HIDDEN), jnp.float32),
          pltpu.VMEM((RPS, HIDDEN), jnp.float32),
          pltpu.VMEM((CH, HIDDEN), jnp.float32),
          pltpu.VMEM((CH, HIDDEN), jnp.float32),
          pltpu.VMEM((CH, HIDDEN), jnp.float32),
          pltpu.VMEM((CH, HIDDEN), jnp.float32),
          pltpu.VMEM_SHARED((NP, 16), jnp.float32),
          pltpu.VMEM_SHARED((NP, HIDDEN), jnp.float32),
          pltpu.VMEM_SHARED((NP, HIDDEN), jnp.float32),
          pltpu.SemaphoreType.DMA,
          pltpu.SemaphoreType.DMA,
          pltpu.SemaphoreType.REGULAR,
      ],
      compiler_params=pltpu.CompilerParams(use_tc_tiling_on_sc=False,
                                           needs_layout_passes=False),
  )
  def k(hw1_hbm, src_hbm, dst_hbm, ones_hbm, zero16_hbm, zero32_hbm,
        p_out, dinv_out, hp_out, q_out,
        deg2_v, ones_v, src_v, dst_v, deg_v, pb2_v, hw_v,
        rows0, rows1, rows2, rows3, acc16, g1t, acc32,
        dsem, gsem, xsem):
    cid = lax.axis_index("c")
    sid = lax.axis_index("s")
    wid = cid * NS + sid
    row0 = sid * RPS

    pltpu.sync_copy(zero16_hbm.at[pl.ds(row0, RPS)], acc16.at[pl.ds(row0, RPS)])
    pltpu.sync_copy(dst_hbm.at[wid], dst_v)
    pltpu.sync_copy(src_hbm.at[wid], src_v)
    pltpu.sync_copy(ones_hbm, ones_v)
    plsc.subcore_barrier()

    def fire(j, carry):
      pltpu.async_copy(ones_v, acc16.at[dst_v.at[j]], dsem, add=True)
      return carry

    lax.fori_loop(0, NCH, fire, 0)

    def drain(j, carry):
      pltpu.make_async_copy(zero16_hbm.at[pl.ds(0, CH)], ones_v, dsem).wait()
      return carry

    lax.fori_loop(0, NCH, drain, 0)
    plsc.subcore_barrier()
    pltpu.sync_copy(acc16.at[pl.ds(row0, RPS)],
                    dinv_out.at[cid, pl.ds(row0, RPS)])
    plsc.subcore_barrier()

    @pl.when(sid == 0)
    def _():
      pl.semaphore_signal(xsem, 1, core_index=1 - cid)
      pl.semaphore_wait(xsem, 1)

    plsc.subcore_barrier()

    pltpu.sync_copy(dinv_out.at[0, pl.ds(row0, RPS)], deg_v)
    pltpu.sync_copy(dinv_out.at[1, pl.ds(row0, RPS)], deg2_v)
    pltpu.sync_copy(hw1_hbm.at[pl.ds(row0, RPS)], hw_v)

    def nrow(r, carry):
      xdeg = deg_v[r] + deg2_v[r] + 1.0
      i = plsc.bitcast(xdeg, jnp.int32)
      i = 0x5F3759DF - lax.shift_right_logical(i, 1)
      y = plsc.bitcast(i, jnp.float32)
      y = y * (1.5 - 0.5 * xdeg * y * y)
      y = y * (1.5 - 0.5 * xdeg * y * y)
      y = y * (1.5 - 0.5 * xdeg * y * y)
      deg_v[r] = y
      hw_v[r, pl.ds(0, 16)] = hw_v[r, pl.ds(0, 16)] * y
      hw_v[r, pl.ds(16, 16)] = hw_v[r, pl.ds(16, 16)] * y
      return carry

    lax.fori_loop(0, RPS, nrow, 0)
    pltpu.sync_copy(deg_v, dinv_out.at[cid, pl.ds(row0, RPS)])
    pltpu.sync_copy(hw_v, g1t.at[pl.ds(row0, RPS)])

    @pl.when(cid == 0)
    def _():
      pltpu.sync_copy(hw_v, acc32.at[pl.ds(row0, RPS)])

    @pl.when(cid == 1)
    def _():
      pltpu.sync_copy(zero32_hbm.at[pl.ds(row0, RPS)],
                      acc32.at[pl.ds(row0, RPS)])

    plsc.subcore_barrier()

    bufs = (rows0, rows1, rows2, rows3)

    def gstart(j, buf):
      pltpu.async_copy(g1t.at[src_v.at[j]], buf, gsem)

    def gwait(buf):
      pltpu.make_async_copy(zero32_hbm.at[pl.ds(0, CH)], buf, gsem).wait()

    def scat(j, buf):
      pltpu.sync_copy(buf, acc32.at[dst_v.at[j]], add=True)

    def segsum_pass():
      gstart(0, bufs[0])
      gstart(1, bufs[1])
      gstart(2, bufs[2])
      gwait(bufs[0])
      gstart(3, bufs[3])
      scat(0, bufs[0])

      def quad(k4, carry):
        j0 = 4 * k4
        for b in (1, 2, 3, 0):
          j = j0 + b if b else j0 + 4
          gwait(bufs[b])
          gstart(j + 3, bufs[(b + 3) % 4])
          scat(j, bufs[b])
        return carry

      lax.fori_loop(0, (NCH - 4) // 4, quad, 0)
      for j, b in ((NCH - 3, 1), (NCH - 2, 2), (NCH - 1, 3)):
        gwait(bufs[b])
        scat(j, bufs[b])

    segsum_pass()
    plsc.subcore_barrier()
    pltpu.sync_copy(acc32.at[pl.ds(row0, RPS)],
                    p_out.at[cid, pl.ds(row0, RPS)])

    plsc.subcore_barrier()

    @pl.when(sid == 0)
    def _():
      pl.semaphore_signal(xsem, 1, core_index=1 - cid)
      pl.semaphore_wait(xsem, 1)

    plsc.subcore_barrier()
    pltpu.sync_copy(p_out.at[0, pl.ds(row0, RPS)], hw_v)
    HALF = RPS // 2
    for h in (0, 1):
      pltpu.sync_copy(p_out.at[1, pl.ds(row0 + h * HALF, HALF)], pb2_v)

      def hrow(r, carry, _h=h):
        rr = _h * HALF + r
        y = deg_v[rr]
        for c in (0, 16):
          srow = (hw_v[rr, pl.ds(c, 16)] + pb2_v[r, pl.ds(c, 16)]) * y
          hw_v[rr, pl.ds(c, 16)] = jnp.maximum(srow, 0.0) * y
        return carry

      lax.fori_loop(0, HALF, hrow, 0)
    pltpu.sync_copy(hw_v, hp_out.at[cid, pl.ds(row0, RPS)])
    pltpu.sync_copy(hw_v, g1t.at[pl.ds(row0, RPS)])
    pltpu.sync_copy(zero32_hbm.at[pl.ds(row0, RPS)],
                    acc32.at[pl.ds(row0, RPS)])
    plsc.subcore_barrier()

    segsum_pass()
    plsc.subcore_barrier()
    pltpu.sync_copy(acc32.at[pl.ds(row0, RPS)],
                    q_out.at[cid, pl.ds(row0, RPS)])

  return k


def _mat1_body(x_ref, w1_ref, out_ref):
  out_ref[...] = jnp.dot(x_ref[...], w1_ref[...],
                         preferred_element_type=jnp.float32)


def _enc2_body(dinv_ref, q_ref, hp_ref, w2_ref, out_ref):
  dinv = dinv_ref[0, :, 0]
  s = (q_ref[0] + q_ref[1] + hp_ref[0]) * dinv[:, None]
  out_ref[...] = jnp.dot(s, w2_ref[...], preferred_element_type=jnp.float32)


_BM = 200


def _dec_body(ei_ref, ej_ref, out_ref):
  z = lax.dot_general(ei_ref[...], ej_ref[...], (((1,), (1,)), ((), ())),
                      preferred_element_type=jnp.float32)
  out_ref[...] = 0.5 * jnp.tanh(z * 0.5) + 0.5


def kernel(x, edge_index, W1, W2):
  src3 = edge_index[0].reshape(NW, NCH, CH)
  dst3 = edge_index[1].reshape(NW, NCH, CH)
  zeros16 = jnp.zeros((NP, 16), jnp.float32)
  zeros32 = jnp.zeros((NP, HIDDEN), jnp.float32)
  ones = jnp.ones((CH, 16), jnp.float32)

  hw1 = pl.pallas_call(
      _mat1_body,
      grid=(N // _RB,),
      in_specs=[
          pl.BlockSpec((_RB, D_FEAT), lambda i: (i, 0)),
          pl.BlockSpec((D_FEAT, HIDDEN), lambda i: (0, 0)),
      ],
      out_specs=pl.BlockSpec((_RB, HIDDEN), lambda i: (i, 0)),
      out_shape=jax.ShapeDtypeStruct((N, HIDDEN), jnp.float32),
  )(x, W1)

  p1, dinv16, hp, q = _make_fused_layer1_kernel()(
      hw1, src3, dst3, ones, zeros16, zeros32)

  encoded = pl.pallas_call(
      _enc2_body,
      grid=(N // _RB,),
      in_specs=[
          pl.BlockSpec((NC, _RB, 16), lambda i: (0, i, 0)),
          pl.BlockSpec((NC, _RB, HIDDEN), lambda i: (0, i, 0)),
          pl.BlockSpec((NC, _RB, HIDDEN), lambda i: (0, i, 0)),
          pl.BlockSpec((HIDDEN, CODE), lambda i: (0, 0)),
      ],
      out_specs=pl.BlockSpec((_RB, CODE), lambda i: (i, 0)),
      out_shape=jax.ShapeDtypeStruct((N, CODE), jnp.float32),
  )(dinv16, q, hp, W2)

  prediction = pl.pallas_call(
      _dec_body,
      grid=(N // _BM,),
      in_specs=[
          pl.BlockSpec((_BM, CODE), lambda i: (i, 0)),
          pl.BlockSpec((N, CODE), lambda i: (0, 0)),
      ],
      out_specs=pl.BlockSpec((_BM, N), lambda i: (i, 0)),
      out_shape=jax.ShapeDtypeStruct((N, N), jnp.float32),
      compiler_params=pltpu.CompilerParams(
          dimension_semantics=("arbitrary",)),
  )(encoded, encoded)

  return prediction


# speedup vs baseline: 1.0725x; 1.0326x over previous
import functools

import jax
import jax.numpy as jnp
from jax import lax
from jax.experimental import pallas as pl
from jax.experimental.pallas import tpu as pltpu
from jax.experimental.pallas import tpu_sc as plsc

N = 10000
D_FEAT = 128
HIDDEN = 32
CODE = 16
E = 160000

NC = 2
NS = 16
NW = NC * NS
EPW = E // NW
CH = 125
NCH = EPW // CH
NP = 10240
RPS = NP // NS

def _mesh():
  return plsc.VectorSubcoreMesh(
      core_axis_name="c", subcore_axis_name="s", num_cores=NC, num_subcores=NS)


@functools.lru_cache(maxsize=None)
def _make_degree_kernel():
  @functools.partial(
      pl.kernel,
      out_type=jax.ShapeDtypeStruct((NC, NP, 16), jnp.float32),
      mesh=_mesh(),
      scratch_types=[
          pltpu.VMEM((NCH, CH), jnp.int32),
          pltpu.VMEM((CH, 16), jnp.float32),
          pltpu.VMEM_SHARED((NP, 16), jnp.float32),
          pltpu.SemaphoreType.DMA,
      ],
      compiler_params=pltpu.CompilerParams(use_tc_tiling_on_sc=False),
  )
  def k(dst_hbm, ones_hbm, zero_hbm, out_hbm, dst_v, ones_v, acc, sem):
    cid = lax.axis_index("c")
    sid = lax.axis_index("s")
    wid = cid * NS + sid
    pltpu.sync_copy(zero_hbm.at[pl.ds(sid * RPS, RPS)],
                    acc.at[pl.ds(sid * RPS, RPS)])
    pltpu.sync_copy(dst_hbm.at[wid], dst_v)
    pltpu.sync_copy(ones_hbm, ones_v)
    plsc.subcore_barrier()

    def fire(j, carry):
      pltpu.async_copy(ones_v, acc.at[dst_v.at[j]], sem, add=True)
      return carry

    lax.fori_loop(0, NCH, fire, 0)

    def drain(j, carry):
      pltpu.make_async_copy(zero_hbm.at[pl.ds(0, CH)], ones_v, sem).wait()
      return carry

    lax.fori_loop(0, NCH, drain, 0)
    plsc.subcore_barrier()
    pltpu.sync_copy(acc.at[pl.ds(sid * RPS, RPS)],
                    out_hbm.at[cid, pl.ds(sid * RPS, RPS)])

  return k


@functools.lru_cache(maxsize=None)
def _make_segsum_kernel(F):
  @functools.partial(
      pl.kernel,
      out_type=jax.ShapeDtypeStruct((NC, NP, F), jnp.float32),
      mesh=_mesh(),
      scratch_types=[
          pltpu.VMEM((NCH, CH), jnp.int32),
          pltpu.VMEM((NCH, CH), jnp.int32),
          pltpu.VMEM((CH, F), jnp.float32),
          pltpu.VMEM((CH, F), jnp.float32),
          pltpu.VMEM((CH, F), jnp.float32),
          pltpu.VMEM((CH, F), jnp.float32),
          pltpu.VMEM_SHARED((NP, F), jnp.float32),
          pltpu.SemaphoreType.DMA,
          pltpu.SemaphoreType.DMA,
      ],
      compiler_params=pltpu.CompilerParams(use_tc_tiling_on_sc=False),
  )
  def k(g_hbm, src_hbm, dst_hbm, zero_hbm, out_hbm,
        src_v, dst_v, rows0, rows1, rows2, rows3, acc, gsem, ssem):
    cid = lax.axis_index("c")
    sid = lax.axis_index("s")
    wid = cid * NS + sid
    pltpu.sync_copy(zero_hbm.at[pl.ds(sid * RPS, RPS)],
                    acc.at[pl.ds(sid * RPS, RPS)])
    pltpu.sync_copy(src_hbm.at[wid], src_v)
    pltpu.sync_copy(dst_hbm.at[wid], dst_v)
    plsc.subcore_barrier()

    bufs = (rows0, rows1, rows2, rows3)

    def gstart(j, buf):
      pltpu.async_copy(g_hbm.at[src_v.at[j]], buf, gsem)

    def gwait(buf):
      pltpu.make_async_copy(g_hbm.at[pl.ds(0, CH)], buf, gsem).wait()

    def scat(j, buf):
      pltpu.sync_copy(buf, acc.at[dst_v.at[j]], add=True)

    gstart(0, bufs[0])
    gstart(1, bufs[1])
    gstart(2, bufs[2])
    gwait(bufs[0])
    gstart(3, bufs[3])
    scat(0, bufs[0])

    def quad(k4, carry):
      j0 = 4 * k4
      for b in (1, 2, 3, 0):
        j = j0 + b if b else j0 + 4
        gwait(bufs[b])
        gstart(j + 3, bufs[(b + 3) % 4])
        scat(j, bufs[b])
      return carry

    lax.fori_loop(0, (NCH - 4) // 4, quad, 0)
    for j, b in ((NCH - 3, 1), (NCH - 2, 2), (NCH - 1, 3)):
      gwait(bufs[b])
      scat(j, bufs[b])
    plsc.subcore_barrier()
    pltpu.sync_copy(acc.at[pl.ds(sid * RPS, RPS)],
                    out_hbm.at[cid, pl.ds(sid * RPS, RPS)])

  return k


_RB = 1000


def _dinv_from(degp_ref):
  deg = degp_ref[0, :, 0] + degp_ref[1, :, 0] + 1.0
  return lax.rsqrt(jnp.maximum(deg, 1.0))


def _prep1_body(degp_ref, x_ref, w1_ref, out_ref):
  dinv = _dinv_from(degp_ref)
  g = jnp.dot(x_ref[...], w1_ref[...], preferred_element_type=jnp.float32)
  out_ref[...] = g * dinv[:, None]


def _prep2_body(degp_ref, g1_ref, p1_ref, w2_ref, out_ref):
  dinv = _dinv_from(degp_ref)
  s = g1_ref[...] + p1_ref[0] + p1_ref[1]
  h = jnp.maximum(s * dinv[:, None], 0.0)
  g2 = jnp.dot(h, w2_ref[...], preferred_element_type=jnp.float32)
  out_ref[...] = g2 * dinv[:, None]


def _enc_body(degp_ref, g2_ref, p2_ref, out_ref):
  dinv = _dinv_from(degp_ref)
  out_ref[...] = (g2_ref[...] + p2_ref[0] + p2_ref[1]) * dinv[:, None]


_BM = 200


def _dec_body(ei_ref, ej_ref, out_ref):
  z = lax.dot_general(ei_ref[...], ej_ref[...], (((1,), (1,)), ((), ())),
                      preferred_element_type=jnp.float32)
  out_ref[...] = 0.5 * jnp.tanh(z * 0.5) + 0.5


def kernel(x, edge_index, W1, W2):
  src3 = edge_index[0].reshape(NW, NCH, CH)
  dst3 = edge_index[1].reshape(NW, NCH, CH)
  zeros16 = jnp.zeros((NP, 16), jnp.float32)
  zeros32 = jnp.zeros((NP, HIDDEN), jnp.float32)
  ones = jnp.ones((CH, 16), jnp.float32)

  degp = _make_degree_kernel()(dst3, ones, zeros16)

  g1 = pl.pallas_call(
      _prep1_body,
      grid=(N // _RB,),
      in_specs=[
          pl.BlockSpec((NC, _RB, 16), lambda i: (0, i, 0)),
          pl.BlockSpec((_RB, D_FEAT), lambda i: (i, 0)),
          pl.BlockSpec((D_FEAT, HIDDEN), lambda i: (0, 0)),
      ],
      out_specs=pl.BlockSpec((_RB, HIDDEN), lambda i: (i, 0)),
      out_shape=jax.ShapeDtypeStruct((N, HIDDEN), jnp.float32),
  )(degp, x, W1)

  p1 = _make_segsum_kernel(HIDDEN)(g1, src3, dst3, zeros32)

  g2 = pl.pallas_call(
      _prep2_body,
      grid=(N // _RB,),
      in_specs=[
          pl.BlockSpec((NC, _RB, 16), lambda i: (0, i, 0)),
          pl.BlockSpec((_RB, HIDDEN), lambda i: (i, 0)),
          pl.BlockSpec((NC, _RB, HIDDEN), lambda i: (0, i, 0)),
          pl.BlockSpec((HIDDEN, CODE), lambda i: (0, 0)),
      ],
      out_specs=pl.BlockSpec((_RB, CODE), lambda i: (i, 0)),
      out_shape=jax.ShapeDtypeStruct((N, CODE), jnp.float32),
  )(degp, g1, p1, W2)

  p2 = _make_segsum_kernel(CODE)(g2, src3, dst3, zeros16)

  encoded = pl.pallas_call(
      _enc_body,
      grid=(N // _RB,),
      in_specs=[
          pl.BlockSpec((NC, _RB, 16), lambda i: (0, i, 0)),
          pl.BlockSpec((_RB, CODE), lambda i: (i, 0)),
          pl.BlockSpec((NC, _RB, CODE), lambda i: (0, i, 0)),
      ],
      out_specs=pl.BlockSpec((_RB, CODE), lambda i: (i, 0)),
      out_shape=jax.ShapeDtypeStruct((N, CODE), jnp.float32),
  )(degp, g2, p2)

  prediction = pl.pallas_call(
      _dec_body,
      grid=(N // _BM,),
      in_specs=[
          pl.BlockSpec((_BM, CODE), lambda i: (i, 0)),
          pl.BlockSpec((N, CODE), lambda i: (0, 0)),
      ],
      out_specs=pl.BlockSpec((_BM, N), lambda i: (i, 0)),
      out_shape=jax.ShapeDtypeStruct((N, N), jnp.float32),
      compiler_params=pltpu.CompilerParams(
          dimension_semantics=("arbitrary",)),
  )(encoded, encoded)

  return prediction
